# baseline (speedup 1.0000x reference)
import functools

import jax
import jax.numpy as jnp
from jax import lax
from jax.experimental import pallas as pl
from jax.experimental.pallas import tpu as pltpu
from jax.experimental.pallas import tpu_sc as plsc

_PB = 205


def _gather_rows_sc(table, ids):
    (batch,) = ids.shape
    _, row_width = table.shape
    mesh = plsc.VectorSubcoreMesh(
        core_axis_name="c", subcore_axis_name="s", num_cores=1
    )

    @functools.partial(
        pl.kernel,
        mesh=mesh,
        out_type=jax.ShapeDtypeStruct((batch, row_width), jnp.float32),
        scratch_types=[
            pltpu.VMEM((batch,), jnp.int32),
            pltpu.VMEM((batch, row_width), jnp.float32),
            pltpu.SemaphoreType.DMA,
        ],
        compiler_params=pltpu.CompilerParams(skip_device_barrier=True),
    )
    def gather_kernel(table_hbm, ids_hbm, out_hbm, idx_v, rows_v, sem):
        wid = lax.axis_index("s") * 2 + lax.axis_index("c")

        @pl.when(wid == 0)
        def _():
            pltpu.sync_copy(ids_hbm, idx_v)
            pltpu.async_copy(table_hbm.at[idx_v], rows_v, sem).wait()
            pltpu.sync_copy(rows_v, out_hbm)

    return gather_kernel(table, ids)


def _add_body(g_ref, h_ref, e_ref, o_ref):
    i = pl.program_id(0) // 5
    t = jnp.tanh(g_ref[...])
    for tile in range(4):
        et = e_ref[pl.ds(i, 1), pl.ds(tile * 1280, 1280)]
        o_ref[0, :, tile, :] = h_ref[0, :, tile, :] + t * et


def kernel(hidden_state, aspect_ratio_ids, embedding_table, gate):
    b, t, p, h = hidden_state.shape
    rows = _gather_rows_sc(embedding_table, aspect_ratio_ids)
    nblk = b * p // _PB
    hid = hidden_state.transpose(0, 2, 1, 3).reshape(nblk, _PB, t, h)
    g = gate.reshape(1, 1)
    out = pl.pallas_call(
        _add_body,
        grid=(nblk,),
        in_specs=[
            pl.BlockSpec((1, 1), lambda i: (0, 0)),
            pl.BlockSpec((1, _PB, t, h), lambda i: (i, 0, 0, 0)),
            pl.BlockSpec((b, t * h), lambda i: (0, 0)),
        ],
        out_specs=pl.BlockSpec((1, _PB, t, h), lambda i: (i, 0, 0, 0)),
        out_shape=jax.ShapeDtypeStruct((nblk, _PB, t, h), jnp.float32),
    )(g, hid, rows)
    return out.reshape(b, p, t, h).transpose(0, 2, 1, 3)


# speedup vs baseline: 1.0312x; 1.0312x over previous
import functools

import jax
import jax.numpy as jnp
from jax import lax
from jax.experimental import pallas as pl
from jax.experimental.pallas import tpu as pltpu
from jax.experimental.pallas import tpu_sc as plsc

_PB = 205


def _gather_rows_sc(table, ids):
    (batch,) = ids.shape
    _, row_width = table.shape
    mesh = plsc.VectorSubcoreMesh(
        core_axis_name="c", subcore_axis_name="s", num_cores=1
    )

    @functools.partial(
        pl.kernel,
        mesh=mesh,
        out_type=jax.ShapeDtypeStruct((batch, row_width), jnp.float32),
        scratch_types=[
            pltpu.VMEM((batch,), jnp.int32),
            pltpu.VMEM((batch, row_width), jnp.float32),
            pltpu.SemaphoreType.DMA,
        ],
    )
    def gather_kernel(table_hbm, ids_hbm, out_hbm, idx_v, rows_v, sem):
        wid = lax.axis_index("s") * 2 + lax.axis_index("c")

        @pl.when(wid == 0)
        def _():
            pltpu.sync_copy(ids_hbm, idx_v)
            pltpu.async_copy(table_hbm.at[idx_v], rows_v, sem).wait()
            pltpu.sync_copy(rows_v, out_hbm)

    return gather_kernel(table, ids)


_NBUF = 3


def _add_body(g_ref, e_ref, h_hbm, o_hbm, ibuf, obuf, isem, osem):
    i = pl.program_id(0)
    n = pl.num_programs(0)
    slot = jax.lax.rem(i, _NBUF)
    bpb = 1025 // _PB

    @pl.when(i == 0)
    def _():
        for k in range(_NBUF):
            pltpu.make_async_copy(h_hbm.at[k], ibuf.at[k], isem.at[k]).start()

    pltpu.make_async_copy(h_hbm.at[i], ibuf.at[slot], isem.at[slot]).wait()

    @pl.when(i >= _NBUF)
    def _():
        pltpu.make_async_copy(obuf.at[slot], o_hbm.at[i], osem.at[slot]).wait()

    bi = i // bpb
    t = jnp.tanh(g_ref[...])
    for tile in range(4):
        et = e_ref[pl.ds(bi, 1), pl.ds(tile * 1280, 1280)]
        obuf[slot, :, tile, :] = ibuf[slot, :, tile, :] + t * et

    pltpu.make_async_copy(obuf.at[slot], o_hbm.at[i], osem.at[slot]).start()

    @pl.when(i + _NBUF < n)
    def _():
        pltpu.make_async_copy(
            h_hbm.at[i + _NBUF], ibuf.at[slot], isem.at[slot]
        ).start()

    @pl.when(i == n - 1)
    def _():
        for k in range(_NBUF):
            s = (n - _NBUF + k) % _NBUF
            pltpu.make_async_copy(obuf.at[s], o_hbm.at[0], osem.at[s]).wait()


def kernel(hidden_state, aspect_ratio_ids, embedding_table, gate):
    b, t, p, h = hidden_state.shape
    rows = _gather_rows_sc(embedding_table, aspect_ratio_ids)
    nblk = b * p // _PB
    hid = hidden_state.transpose(0, 2, 1, 3).reshape(nblk, _PB, t, h)
    g = gate.reshape(1, 1)
    out = pl.pallas_call(
        _add_body,
        grid=(nblk,),
        in_specs=[
            pl.BlockSpec((1, 1), lambda i: (0, 0)),
            pl.BlockSpec((b, t * h), lambda i: (0, 0)),
            pl.BlockSpec(memory_space=pltpu.MemorySpace.HBM),
        ],
        out_specs=pl.BlockSpec(memory_space=pltpu.MemorySpace.HBM),
        out_shape=jax.ShapeDtypeStruct((nblk, _PB, t, h), jnp.float32),
        scratch_shapes=[
            pltpu.VMEM((_NBUF, _PB, t, h), jnp.float32),
            pltpu.VMEM((_NBUF, _PB, t, h), jnp.float32),
            pltpu.SemaphoreType.DMA((_NBUF,)),
            pltpu.SemaphoreType.DMA((_NBUF,)),
        ],
    )(g, rows, hid)
    return out.reshape(b, p, t, h).transpose(0, 2, 1, 3)
